# Initial kernel scaffold; baseline (speedup 1.0000x reference)
import jax
import jax.numpy as jnp
from jax.experimental import pallas as pl


def kernel(angles, sequence, W_e0, b_e0, W_e1, b_e1, W_e2, b_e2, W_d0, b_d0, W_d1, b_d1, W_d2, b_d2, codebook):
    raise NotImplementedError("write your pallas kernel here")


# speedup vs baseline: 1.3061x; 1.3061x over previous
import functools

import jax
import jax.numpy as jnp
from jax import lax
from jax.experimental import pallas as pl
from jax.experimental.pallas import tpu as pltpu
from jax.experimental.pallas import tpu_sc as plsc

A = 128
B = 1024
HID = 1024
EMB = 512
K = 8192
UC = 768
NAA = 2560
IN_AUG = UC + NAA

BB = 256
KB = 512

f32 = jnp.float32
bf16 = jnp.bfloat16


def _dgT(x, y):
    return lax.dot_general(x, y, (((1,), (1,)), ((), ())),
                           preferred_element_type=f32)


def _enc_body(ang_ref, wc_ref, ws_ref, b0_ref, w1_ref, b1_ref, w2_ref, b2_ref,
              c_ref, s_ref, e_ref):
    a = ang_ref[...]
    c = jnp.cos(a)
    s = jnp.sin(a)
    c_ref[...] = c
    s_ref[...] = s
    h = jnp.dot(c.astype(bf16), wc_ref[...], preferred_element_type=f32)
    h = h + jnp.dot(s.astype(bf16), ws_ref[...], preferred_element_type=f32)
    h = jnp.maximum(h + b0_ref[...], 0.0)
    h = jnp.maximum(
        jnp.dot(h.astype(bf16), w1_ref[...], preferred_element_type=f32)
        + b1_ref[...], 0.0)
    e_ref[...] = jnp.maximum(
        jnp.dot(h.astype(bf16), w2_ref[...], preferred_element_type=f32)
        + b2_ref[...], 0.0)


def _cn_body(cb_ref, cn_ref):
    c = cb_ref[...].astype(f32)
    cn_ref[...] = jnp.sum(c * c, axis=1, keepdims=True)


def _vq_body(e_ref, cb_ref, cn_ref, bv_ref, bi_ref):
    k = pl.program_id(1)
    s = cn_ref[...] - 2.0 * _dgT(cb_ref[...], e_ref[...])
    lm = jnp.min(s, axis=0)
    ii = lax.broadcasted_iota(jnp.int32, s.shape, 0) + k * KB
    la = jnp.min(jnp.where(s == lm[None, :], ii, K), axis=0)

    @pl.when(k == 0)
    def _():
        bv_ref[...] = lm[None, :]
        bi_ref[...] = la[None, :]

    @pl.when(k != 0)
    def _():
        pv = bv_ref[...]
        upd = lm[None, :] < pv
        bv_ref[...] = jnp.where(upd, lm[None, :], pv)
        bi_ref[...] = jnp.where(upd, la[None, :], bi_ref[...])


def _gram_body(ci_ref, si_ref, ei_ref, cj_ref, sj_ref, ej_ref,
               num_ref, wm_ref):
    first = (pl.program_id(0) == 0) & (pl.program_id(1) == 0)
    Ci = ci_ref[...]
    Si = si_ref[...]
    Cj = cj_ref[...]
    Sj = sj_ref[...]
    Ei = ei_ref[...]
    Ej = ej_ref[...]
    mc = (_dgT(Ci.astype(bf16), Cj.astype(bf16))
          + _dgT(Si.astype(bf16), Sj.astype(bf16))) * (1.0 / 384.0)
    Dd = 0.5 * (1.0 - mc)
    Y = jnp.where(Dd < 0.1, 1.0, jnp.where(Dd > 0.47, 0.0, 0.5))
    wm = jnp.where(Y == 0.5, 0.0, 1.0)
    e2i = jnp.sum(Ei * Ei, axis=1)
    e2j = jnp.sum(Ej * Ej, axis=1)
    d2 = jnp.maximum(
        e2i[:, None] + e2j[None, :]
        - 2.0 * _dgT(Ei.astype(bf16), Ej.astype(bf16)), 0.0)
    dn = jnp.sqrt(d2 + 1e-8)
    rl = jnp.maximum(1.0 - dn, 0.0)
    t = wm * (Y * (d2 + 1e-8) + (1.0 - Y) * rl * rl)
    sm = jnp.sum(t).reshape(1, 1)
    wsm = jnp.sum(wm).reshape(1, 1)

    @pl.when(first)
    def _():
        num_ref[...] = sm
        wm_ref[...] = wsm

    @pl.when(~first)
    def _():
        num_ref[...] = num_ref[...] + sm
        wm_ref[...] = wm_ref[...] + wsm


def _dec_body(q_ref, e_ref, c_ref, s_ref, seq_ref,
              w0_ref, b0_ref, w1_ref, b1_ref, w2_ref, b2_ref,
              rec_ref, ucr_ref, com_ref, aa_ref):
    first = pl.program_id(0) == 0
    q = q_ref[...]
    h = jnp.maximum(
        jnp.dot(q.astype(bf16), w0_ref[...], preferred_element_type=f32)
        + b0_ref[...], 0.0)
    h = jnp.maximum(
        jnp.dot(h.astype(bf16), w1_ref[...], preferred_element_type=f32)
        + b1_ref[...], 0.0)
    dec = (jnp.dot(h.astype(bf16), w2_ref[...], preferred_element_type=f32)
           + b2_ref[...])
    C = c_ref[...]
    S = s_ref[...]
    xr = dec[:, 0:384]
    yr = dec[:, 384:768]
    aa = dec[:, 768:]
    rec = (jnp.sum((xr - C) ** 2) + jnp.sum((yr - S) ** 2)
           + jnp.sum(aa * aa))
    r2 = xr * xr + yr * yr
    ucr = jnp.sum((r2 - 1.0) ** 2)
    com = jnp.sum((e_ref[...] - q) ** 2)
    seq = seq_ref[...]
    m = aa[:, 0:A]
    best = seq[:, 0:A]
    sel = aa[:, 0:A]
    for c in range(1, 20):
        Lc = aa[:, c * A:(c + 1) * A]
        sc = seq[:, c * A:(c + 1) * A]
        m = jnp.maximum(m, Lc)
        upd = sc > best
        best = jnp.where(upd, sc, best)
        sel = jnp.where(upd, Lc, sel)
    se = jnp.zeros_like(m)
    for c in range(20):
        se = se + jnp.exp(aa[:, c * A:(c + 1) * A] - m)
    lse = m + jnp.log(se)
    aas = jnp.sum(sel - lse)

    rec = rec.reshape(1, 1)
    ucr = ucr.reshape(1, 1)
    com = com.reshape(1, 1)
    aas = aas.reshape(1, 1)

    @pl.when(first)
    def _():
        rec_ref[...] = rec
        ucr_ref[...] = ucr
        com_ref[...] = com
        aa_ref[...] = aas

    @pl.when(~first)
    def _():
        rec_ref[...] = rec_ref[...] + rec
        ucr_ref[...] = ucr_ref[...] + ucr
        com_ref[...] = com_ref[...] + com
        aa_ref[...] = aa_ref[...] + aas


def _whole(shape):
    return pl.BlockSpec(shape, lambda *_: tuple(0 for _ in shape))


def _sc_gather(table, idx):
    info = plsc.get_sparse_core_info()
    nw = info.num_cores * info.num_subcores
    bpw = B // nw
    mesh = plsc.VectorSubcoreMesh(core_axis_name="c", subcore_axis_name="s")

    @functools.partial(
        pl.kernel, mesh=mesh,
        out_type=jax.ShapeDtypeStruct((B, EMB), f32),
        scratch_types=[
            pltpu.VMEM((bpw,), jnp.int32),
            pltpu.VMEM((bpw, EMB), f32),
            pltpu.SemaphoreType.DMA,
        ],
    )
    def k(table_hbm, idx_hbm, out_hbm, idx_v, rows_v, sem):
        wid = lax.axis_index("s") * info.num_cores + lax.axis_index("c")
        base = wid * bpw
        pltpu.sync_copy(idx_hbm.at[pl.ds(base, bpw)], idx_v)
        pltpu.async_copy(table_hbm.at[idx_v], rows_v, sem).wait()
        pltpu.sync_copy(rows_v, out_hbm.at[pl.ds(base, bpw)])

    return k(table, idx)


def kernel(angles, sequence, W_e0, b_e0, W_e1, b_e1, W_e2, b_e2,
           W_d0, b_d0, W_d1, b_d1, W_d2, b_d2, codebook):
    nb = B // BB
    Wc = W_e0[0:UC:2].astype(bf16)
    Ws = W_e0[1:UC:2].astype(bf16)
    w1 = W_e1.astype(bf16)
    w2 = W_e2.astype(bf16)
    wd0 = W_d0.astype(bf16)
    wd1 = W_d1.astype(bf16)
    Wd2p = jnp.concatenate(
        [W_d2[:, 0:UC:2], W_d2[:, 1:UC:2],
         W_d2[:, UC:].reshape(HID, A, 20).transpose(0, 2, 1).reshape(HID, NAA)],
        axis=1).astype(bf16)
    bd2p = jnp.concatenate(
        [b_d2[0:UC:2], b_d2[1:UC:2],
         b_d2[UC:].reshape(A, 20).T.reshape(-1)]).reshape(1, IN_AUG)
    seqp = sequence.reshape(B, A, 20).transpose(0, 2, 1).reshape(B, NAA)
    cb_bf = codebook.astype(bf16)
    b0 = b_e0.reshape(1, HID)
    b1 = b_e1.reshape(1, HID)
    b2 = b_e2.reshape(1, EMB)
    bd0 = b_d0.reshape(1, HID)
    bd1 = b_d1.reshape(1, HID)

    C, S, enc = pl.pallas_call(
        _enc_body,
        grid=(nb,),
        in_specs=[
            pl.BlockSpec((BB, 384), lambda i: (i, 0)),
            _whole((384, HID)), _whole((384, HID)), _whole((1, HID)),
            _whole((HID, HID)), _whole((1, HID)),
            _whole((HID, EMB)), _whole((1, EMB)),
        ],
        out_specs=[
            pl.BlockSpec((BB, 384), lambda i: (i, 0)),
            pl.BlockSpec((BB, 384), lambda i: (i, 0)),
            pl.BlockSpec((BB, EMB), lambda i: (i, 0)),
        ],
        out_shape=[
            jax.ShapeDtypeStruct((B, 384), f32),
            jax.ShapeDtypeStruct((B, 384), f32),
            jax.ShapeDtypeStruct((B, EMB), f32),
        ],
    )(angles, Wc, Ws, b0, w1, b1, w2, b2)
    enc_bf = enc.astype(bf16)

    cn = pl.pallas_call(
        _cn_body,
        grid=(K // 1024,),
        in_specs=[pl.BlockSpec((1024, EMB), lambda k: (k, 0))],
        out_specs=pl.BlockSpec((1024, 1), lambda k: (k, 0)),
        out_shape=jax.ShapeDtypeStruct((K, 1), f32),
    )(cb_bf)
    _, bi = pl.pallas_call(
        _vq_body,
        grid=(nb, K // KB),
        in_specs=[
            pl.BlockSpec((BB, EMB), lambda b, k: (b, 0)),
            pl.BlockSpec((KB, EMB), lambda b, k: (k, 0)),
            pl.BlockSpec((KB, 1), lambda b, k: (k, 0)),
        ],
        out_specs=[
            pl.BlockSpec((1, BB), lambda b, k: (0, b)),
            pl.BlockSpec((1, BB), lambda b, k: (0, b)),
        ],
        out_shape=[
            jax.ShapeDtypeStruct((1, B), f32),
            jax.ShapeDtypeStruct((1, B), jnp.int32),
        ],
    )(enc_bf, cb_bf, cn)

    quant = _sc_gather(codebook, bi[0, :])

    num, wsum = pl.pallas_call(
        _gram_body,
        grid=(nb, nb),
        in_specs=[
            pl.BlockSpec((BB, 384), lambda i, j: (i, 0)),
            pl.BlockSpec((BB, 384), lambda i, j: (i, 0)),
            pl.BlockSpec((BB, EMB), lambda i, j: (i, 0)),
            pl.BlockSpec((BB, 384), lambda i, j: (j, 0)),
            pl.BlockSpec((BB, 384), lambda i, j: (j, 0)),
            pl.BlockSpec((BB, EMB), lambda i, j: (j, 0)),
        ],
        out_specs=[
            pl.BlockSpec((1, 1), lambda i, j: (0, 0)),
            pl.BlockSpec((1, 1), lambda i, j: (0, 0)),
        ],
        out_shape=[
            jax.ShapeDtypeStruct((1, 1), f32),
            jax.ShapeDtypeStruct((1, 1), f32),
        ],
    )(C, S, enc, C, S, enc)

    rec, ucr, com, aas = pl.pallas_call(
        _dec_body,
        grid=(nb,),
        in_specs=[
            pl.BlockSpec((BB, EMB), lambda i: (i, 0)),
            pl.BlockSpec((BB, EMB), lambda i: (i, 0)),
            pl.BlockSpec((BB, 384), lambda i: (i, 0)),
            pl.BlockSpec((BB, 384), lambda i: (i, 0)),
            pl.BlockSpec((BB, NAA), lambda i: (i, 0)),
            _whole((EMB, HID)), _whole((1, HID)),
            _whole((HID, HID)), _whole((1, HID)),
            _whole((HID, IN_AUG)), _whole((1, IN_AUG)),
        ],
        out_specs=[pl.BlockSpec((1, 1), lambda i: (0, 0))] * 4,
        out_shape=[jax.ShapeDtypeStruct((1, 1), f32)] * 4,
    )(quant, enc, C, S, seqp, wd0, bd0, wd1, bd1, Wd2p, bd2p)

    recon = rec[0, 0] / (B * IN_AUG)
    commit = 0.25 * com[0, 0] / (B * EMB)
    aa_loss = -aas[0, 0] / (B * A)
    uc_reg = ucr[0, 0] / (B * 384)
    dih = num[0, 0] / jnp.maximum(wsum[0, 0], 1.0)
    return recon + commit + aa_loss + 0.01 * uc_reg + 0.1 * dih


# speedup vs baseline: 1.4680x; 1.1240x over previous
import functools

import jax
import jax.numpy as jnp
from jax import lax
from jax.experimental import pallas as pl
from jax.experimental.pallas import tpu as pltpu
from jax.experimental.pallas import tpu_sc as plsc

A = 128
B = 1024
HID = 1024
EMB = 512
K = 8192
UC = 768
NAA = 2560
IN_AUG = UC + NAA

BB = 256
KB = 512

f32 = jnp.float32
bf16 = jnp.bfloat16


def _dgT(x, y):
    return lax.dot_general(x, y, (((1,), (1,)), ((), ())),
                           preferred_element_type=f32)


def _encvq_body(ang_ref, wc_ref, ws_ref, b0_ref, w1_ref, b1_ref, w2_ref,
                b2_ref, cb_ref, c_ref, s_ref, e_ref, cb16_ref, sb16_ref,
                eb16_ref, bi_ref):
    a = ang_ref[...]
    c = jnp.cos(a)
    s = jnp.sin(a)
    c_ref[...] = c
    s_ref[...] = s
    cb_ = c.astype(bf16)
    sb_ = s.astype(bf16)
    cb16_ref[...] = cb_
    sb16_ref[...] = sb_
    h = jnp.dot(cb_, wc_ref[...], preferred_element_type=f32)
    h = h + jnp.dot(sb_, ws_ref[...], preferred_element_type=f32)
    h = jnp.maximum(h + b0_ref[...], 0.0)
    h = jnp.maximum(
        jnp.dot(h.astype(bf16), w1_ref[...], preferred_element_type=f32)
        + b1_ref[...], 0.0)
    e = jnp.maximum(
        jnp.dot(h.astype(bf16), w2_ref[...], preferred_element_type=f32)
        + b2_ref[...], 0.0)
    e_ref[...] = e
    eb = e.astype(bf16)
    eb16_ref[...] = eb

    ii = lax.broadcasted_iota(jnp.int32, (KB, e.shape[0]), 0)

    def chunk(k, carry):
        lm, la = carry
        ck = cb_ref[pl.ds(k * KB, KB), :]
        ckf = ck.astype(f32)
        cn = jnp.sum(ckf * ckf, axis=1, keepdims=True)
        sgn = cn - 2.0 * _dgT(ck, eb)
        cm = jnp.min(sgn, axis=0)
        cidx = jnp.min(jnp.where(sgn == cm[None, :], ii + k * KB, K), axis=0)
        upd = cm < lm
        return jnp.where(upd, cm, lm), jnp.where(upd, cidx, la)

    init = (jnp.full((e.shape[0],), jnp.inf, f32),
            jnp.zeros((e.shape[0],), jnp.int32))
    _, la = lax.fori_loop(0, K // KB, chunk, init)
    bi_ref[...] = la[None, :]


def _pair_ij(g):
    i = ((g >= 4).astype(jnp.int32) + (g >= 7).astype(jnp.int32)
         + (g >= 9).astype(jnp.int32))
    offs = jnp.where(i == 0, 0, jnp.where(i == 1, 4, jnp.where(i == 2, 7, 9)))
    j = g - offs + i
    return i, j


def _gram_body(ci_ref, si_ref, ei_ref, cj_ref, sj_ref, ej_ref,
               num_ref, wm_ref):
    g = pl.program_id(0)
    first = g == 0
    gi, gj = _pair_ij(g)
    fac = jnp.where(gi == gj, 1.0, 2.0).astype(f32)
    Ci = ci_ref[...]
    Si = si_ref[...]
    Cj = cj_ref[...]
    Sj = sj_ref[...]
    Ei = ei_ref[...]
    Ej = ej_ref[...]
    mc = (_dgT(Ci, Cj) + _dgT(Si, Sj)) * (1.0 / 384.0)
    Dd = 0.5 * (1.0 - mc)
    Y = jnp.where(Dd < 0.1, 1.0, jnp.where(Dd > 0.47, 0.0, 0.5))
    wm = jnp.where(Y == 0.5, 0.0, 1.0)
    Eif = Ei.astype(f32)
    Ejf = Ej.astype(f32)
    e2i = jnp.sum(Eif * Eif, axis=1)
    e2j = jnp.sum(Ejf * Ejf, axis=1)
    d2 = jnp.maximum(
        e2i[:, None] + e2j[None, :] - 2.0 * _dgT(Ei, Ej), 0.0)
    dn = jnp.sqrt(d2 + 1e-8)
    rl = jnp.maximum(1.0 - dn, 0.0)
    t = wm * (Y * (d2 + 1e-8) + (1.0 - Y) * rl * rl)
    sm = (fac * jnp.sum(t)).reshape(1, 1)
    wsm = (fac * jnp.sum(wm)).reshape(1, 1)

    @pl.when(first)
    def _():
        num_ref[...] = sm
        wm_ref[...] = wsm

    @pl.when(~first)
    def _():
        num_ref[...] = num_ref[...] + sm
        wm_ref[...] = wm_ref[...] + wsm


def _dec_body(q_ref, e_ref, c_ref, s_ref, seq_ref,
              w0_ref, b0_ref, w1_ref, b1_ref, w2_ref, b2_ref,
              rec_ref, ucr_ref, com_ref, aa_ref):
    first = pl.program_id(0) == 0
    q = q_ref[...]
    h = jnp.maximum(
        jnp.dot(q.astype(bf16), w0_ref[...], preferred_element_type=f32)
        + b0_ref[...], 0.0)
    h = jnp.maximum(
        jnp.dot(h.astype(bf16), w1_ref[...], preferred_element_type=f32)
        + b1_ref[...], 0.0)
    dec = (jnp.dot(h.astype(bf16), w2_ref[...], preferred_element_type=f32)
           + b2_ref[...])
    C = c_ref[...]
    S = s_ref[...]
    xr = dec[:, 0:384]
    yr = dec[:, 384:768]
    aa = dec[:, 768:]
    rec = (jnp.sum((xr - C) ** 2) + jnp.sum((yr - S) ** 2)
           + jnp.sum(aa * aa))
    r2 = xr * xr + yr * yr
    ucr = jnp.sum((r2 - 1.0) ** 2)
    com = jnp.sum((e_ref[...] - q) ** 2)
    seq = seq_ref[...]
    m = aa[:, 0:A]
    best = seq[:, 0:A]
    sel = aa[:, 0:A]
    for c in range(1, 20):
        Lc = aa[:, c * A:(c + 1) * A]
        sc = seq[:, c * A:(c + 1) * A]
        m = jnp.maximum(m, Lc)
        upd = sc > best
        best = jnp.where(upd, sc, best)
        sel = jnp.where(upd, Lc, sel)
    se = jnp.zeros_like(m)
    for c in range(20):
        se = se + jnp.exp(aa[:, c * A:(c + 1) * A] - m)
    lse = m + jnp.log(se)
    aas = jnp.sum(sel - lse)

    rec = rec.reshape(1, 1)
    ucr = ucr.reshape(1, 1)
    com = com.reshape(1, 1)
    aas = aas.reshape(1, 1)

    @pl.when(first)
    def _():
        rec_ref[...] = rec
        ucr_ref[...] = ucr
        com_ref[...] = com
        aa_ref[...] = aas

    @pl.when(~first)
    def _():
        rec_ref[...] = rec_ref[...] + rec
        ucr_ref[...] = ucr_ref[...] + ucr
        com_ref[...] = com_ref[...] + com
        aa_ref[...] = aa_ref[...] + aas


def _whole(shape):
    return pl.BlockSpec(shape, lambda *_: tuple(0 for _ in shape))


def _sc_gather(table, idx):
    info = plsc.get_sparse_core_info()
    nw = info.num_cores * info.num_subcores
    bpw = B // nw
    mesh = plsc.VectorSubcoreMesh(core_axis_name="c", subcore_axis_name="s")

    @functools.partial(
        pl.kernel, mesh=mesh,
        out_type=jax.ShapeDtypeStruct((B, EMB), f32),
        scratch_types=[
            pltpu.VMEM((bpw,), jnp.int32),
            pltpu.VMEM((bpw, EMB), f32),
            pltpu.SemaphoreType.DMA,
        ],
    )
    def k(table_hbm, idx_hbm, out_hbm, idx_v, rows_v, sem):
        wid = lax.axis_index("s") * info.num_cores + lax.axis_index("c")
        base = wid * bpw
        pltpu.sync_copy(idx_hbm.at[pl.ds(base, bpw)], idx_v)
        pltpu.async_copy(table_hbm.at[idx_v], rows_v, sem).wait()
        pltpu.sync_copy(rows_v, out_hbm.at[pl.ds(base, bpw)])

    return k(table, idx)


def kernel(angles, sequence, W_e0, b_e0, W_e1, b_e1, W_e2, b_e2,
           W_d0, b_d0, W_d1, b_d1, W_d2, b_d2, codebook):
    nb = B // BB
    Wc = W_e0[0:UC:2].astype(bf16)
    Ws = W_e0[1:UC:2].astype(bf16)
    w1 = W_e1.astype(bf16)
    w2 = W_e2.astype(bf16)
    wd0 = W_d0.astype(bf16)
    wd1 = W_d1.astype(bf16)
    Wd2p = jnp.concatenate(
        [W_d2[:, 0:UC:2], W_d2[:, 1:UC:2],
         W_d2[:, UC:].reshape(HID, A, 20).transpose(0, 2, 1).reshape(HID, NAA)],
        axis=1).astype(bf16)
    bd2p = jnp.concatenate(
        [b_d2[0:UC:2], b_d2[1:UC:2],
         b_d2[UC:].reshape(A, 20).T.reshape(-1)]).reshape(1, IN_AUG)
    seqp = sequence.reshape(B, A, 20).transpose(0, 2, 1).reshape(B, NAA)
    cb_bf = codebook.astype(bf16)
    b0 = b_e0.reshape(1, HID)
    b1 = b_e1.reshape(1, HID)
    b2 = b_e2.reshape(1, EMB)
    bd0 = b_d0.reshape(1, HID)
    bd1 = b_d1.reshape(1, HID)

    C, S, enc, C16, S16, E16, bi = pl.pallas_call(
        _encvq_body,
        grid=(nb,),
        in_specs=[
            pl.BlockSpec((BB, 384), lambda i: (i, 0)),
            _whole((384, HID)), _whole((384, HID)), _whole((1, HID)),
            _whole((HID, HID)), _whole((1, HID)),
            _whole((HID, EMB)), _whole((1, EMB)),
            _whole((K, EMB)),
        ],
        out_specs=[
            pl.BlockSpec((BB, 384), lambda i: (i, 0)),
            pl.BlockSpec((BB, 384), lambda i: (i, 0)),
            pl.BlockSpec((BB, EMB), lambda i: (i, 0)),
            pl.BlockSpec((BB, 384), lambda i: (i, 0)),
            pl.BlockSpec((BB, 384), lambda i: (i, 0)),
            pl.BlockSpec((BB, EMB), lambda i: (i, 0)),
            pl.BlockSpec((1, BB), lambda i: (0, i)),
        ],
        out_shape=[
            jax.ShapeDtypeStruct((B, 384), f32),
            jax.ShapeDtypeStruct((B, 384), f32),
            jax.ShapeDtypeStruct((B, EMB), f32),
            jax.ShapeDtypeStruct((B, 384), bf16),
            jax.ShapeDtypeStruct((B, 384), bf16),
            jax.ShapeDtypeStruct((B, EMB), bf16),
            jax.ShapeDtypeStruct((1, B), jnp.int32),
        ],
    )(angles, Wc, Ws, b0, w1, b1, w2, b2, cb_bf)

    quant = _sc_gather(codebook, bi[0, :])

    def _pi(g):
        return _pair_ij(g)[0]

    def _pj(g):
        return _pair_ij(g)[1]

    num, wsum = pl.pallas_call(
        _gram_body,
        grid=(10,),
        in_specs=[
            pl.BlockSpec((BB, 384), lambda g: (_pi(g), 0)),
            pl.BlockSpec((BB, 384), lambda g: (_pi(g), 0)),
            pl.BlockSpec((BB, EMB), lambda g: (_pi(g), 0)),
            pl.BlockSpec((BB, 384), lambda g: (_pj(g), 0)),
            pl.BlockSpec((BB, 384), lambda g: (_pj(g), 0)),
            pl.BlockSpec((BB, EMB), lambda g: (_pj(g), 0)),
        ],
        out_specs=[
            pl.BlockSpec((1, 1), lambda g: (0, 0)),
            pl.BlockSpec((1, 1), lambda g: (0, 0)),
        ],
        out_shape=[
            jax.ShapeDtypeStruct((1, 1), f32),
            jax.ShapeDtypeStruct((1, 1), f32),
        ],
    )(C16, S16, E16, C16, S16, E16)

    rec, ucr, com, aas = pl.pallas_call(
        _dec_body,
        grid=(nb,),
        in_specs=[
            pl.BlockSpec((BB, EMB), lambda i: (i, 0)),
            pl.BlockSpec((BB, EMB), lambda i: (i, 0)),
            pl.BlockSpec((BB, 384), lambda i: (i, 0)),
            pl.BlockSpec((BB, 384), lambda i: (i, 0)),
            pl.BlockSpec((BB, NAA), lambda i: (i, 0)),
            _whole((EMB, HID)), _whole((1, HID)),
            _whole((HID, HID)), _whole((1, HID)),
            _whole((HID, IN_AUG)), _whole((1, IN_AUG)),
        ],
        out_specs=[pl.BlockSpec((1, 1), lambda i: (0, 0))] * 4,
        out_shape=[jax.ShapeDtypeStruct((1, 1), f32)] * 4,
    )(quant, enc, C, S, seqp, wd0, bd0, wd1, bd1, Wd2p, bd2p)

    recon = rec[0, 0] / (B * IN_AUG)
    commit = 0.25 * com[0, 0] / (B * EMB)
    aa_loss = -aas[0, 0] / (B * A)
    uc_reg = ucr[0, 0] / (B * 384)
    dih = num[0, 0] / jnp.maximum(wsum[0, 0], 1.0)
    return recon + commit + aa_loss + 0.01 * uc_reg + 0.1 * dih
